# baseline (speedup 1.0000x reference)
import functools

import jax
import jax.numpy as jnp
from jax import lax
from jax.experimental import pallas as pl
from jax.experimental.pallas import tpu as pltpu
from jax.experimental.pallas import tpu_sc as plsc

MAX_LEN = 512
D_MODEL = 64
BATCH = 16384

_NUM_CORES = 1
_NUM_SUBCORES = 16
_NUM_WORKERS = _NUM_CORES * _NUM_SUBCORES
_B_PER_W = BATCH // _NUM_WORKERS

_CHUNKS = 2
_C = _B_PER_W // _CHUNKS

_mesh = plsc.VectorSubcoreMesh(
    core_axis_name="c", subcore_axis_name="s",
    num_cores=_NUM_CORES, num_subcores=_NUM_SUBCORES,
)


@functools.partial(
    pl.kernel,
    out_type=jax.ShapeDtypeStruct((BATCH, 128), jnp.float32),
    mesh=_mesh,
    compiler_params=pltpu.CompilerParams(use_tc_tiling_on_sc=False),
    scratch_types=[
        pltpu.VMEM((_B_PER_W,), jnp.int32),
        pltpu.VMEM((_CHUNKS, _C, D_MODEL), jnp.float32),
        [pltpu.SemaphoreType.DMA] * _CHUNKS,
        [pltpu.SemaphoreType.DMA] * _CHUNKS,
    ],
)
def _sc_gather(table_hbm, idx_hbm, out_hbm, idx_v, rows_v, gsems, wsems):
    wid = lax.axis_index("s") * _NUM_CORES + lax.axis_index("c")
    base = wid * _B_PER_W
    pltpu.sync_copy(idx_hbm.at[pl.ds(base, _B_PER_W)], idx_v)
    gathers = [
        pltpu.async_copy(
            table_hbm.at[idx_v.at[pl.ds(c * _C, _C)]], rows_v.at[c], gsems[c]
        )
        for c in range(_CHUNKS)
    ]
    writes = []
    for c in range(_CHUNKS):
        gathers[c].wait()
        writes.append(
            pltpu.async_copy(
                rows_v.at[c],
                out_hbm.at[pl.ds(base + c * _C, _C), pl.ds(0, D_MODEL)],
                wsems[c],
            )
        )
    for w in writes:
        w.wait()


_TC_ROWS = 512


def _xpose_body(in_ref, out_ref):
    out_ref[...] = in_ref[:, :D_MODEL].T


_xpose = pl.pallas_call(
    _xpose_body,
    grid=(BATCH // _TC_ROWS,),
    in_specs=[pl.BlockSpec((_TC_ROWS, 128), lambda i: (i, 0))],
    out_specs=pl.BlockSpec((D_MODEL, _TC_ROWS), lambda i: (0, i)),
    out_shape=jax.ShapeDtypeStruct((D_MODEL, BATCH), jnp.float32),
)


def kernel(positions, table):
    staged = _sc_gather(table, positions.astype(jnp.int32))
    return _xpose(staged).T


# speedup vs baseline: 1.4300x; 1.4300x over previous
import functools

import jax
import jax.numpy as jnp
from jax import lax
from jax.experimental import pallas as pl
from jax.experimental.pallas import tpu as pltpu
from jax.experimental.pallas import tpu_sc as plsc

MAX_LEN = 512
D_MODEL = 64
BATCH = 16384

_NUM_CORES = 1
_NUM_SUBCORES = 16
_NUM_WORKERS = _NUM_CORES * _NUM_SUBCORES
_B_PER_W = BATCH // _NUM_WORKERS

_CHUNKS = 2
_C = _B_PER_W // _CHUNKS

_mesh = plsc.VectorSubcoreMesh(
    core_axis_name="c", subcore_axis_name="s",
    num_cores=_NUM_CORES, num_subcores=_NUM_SUBCORES,
)


@functools.partial(
    pl.kernel,
    out_type=jax.ShapeDtypeStruct((BATCH, 128), jnp.float32),
    mesh=_mesh,
    compiler_params=pltpu.CompilerParams(use_tc_tiling_on_sc=False),
    scratch_types=[
        pltpu.VMEM((_B_PER_W,), jnp.int32),
        pltpu.VMEM((_CHUNKS, _C, D_MODEL), jnp.float32),
        [pltpu.SemaphoreType.DMA] * _CHUNKS,
        [pltpu.SemaphoreType.DMA] * _CHUNKS,
    ],
)
def _sc_gather(table_hbm, idx_hbm, out_hbm, idx_v, rows_v, gsems, wsems):
    wid = lax.axis_index("s") * _NUM_CORES + lax.axis_index("c")
    base = wid * _B_PER_W
    pltpu.sync_copy(idx_hbm.at[pl.ds(base, _B_PER_W)], idx_v)
    gathers = [
        pltpu.async_copy(
            table_hbm.at[idx_v.at[pl.ds(c * _C, _C)]], rows_v.at[c], gsems[c]
        )
        for c in range(_CHUNKS)
    ]
    writes = []
    for c in range(_CHUNKS):
        gathers[c].wait()
        writes.append(
            pltpu.async_copy(
                rows_v.at[c],
                out_hbm.at[pl.ds(base + c * _C, _C), pl.ds(0, D_MODEL)],
                wsems[c],
            )
        )
    for w in writes:
        w.wait()


_TC_ROWS = 4096


def _xpose_body(in_ref, out_ref):
    out_ref[...] = in_ref[:, :D_MODEL].T


_xpose = pl.pallas_call(
    _xpose_body,
    grid=(BATCH // _TC_ROWS,),
    in_specs=[pl.BlockSpec((_TC_ROWS, 128), lambda i: (i, 0))],
    out_specs=pl.BlockSpec((D_MODEL, _TC_ROWS), lambda i: (0, i)),
    out_shape=jax.ShapeDtypeStruct((D_MODEL, BATCH), jnp.float32),
)


def kernel(positions, table):
    staged = _sc_gather(table, positions.astype(jnp.int32))
    return _xpose(staged).T


# speedup vs baseline: 1.4834x; 1.0374x over previous
import functools

import jax
import jax.numpy as jnp
from jax import lax
from jax.experimental import pallas as pl
from jax.experimental.pallas import tpu as pltpu
from jax.experimental.pallas import tpu_sc as plsc

MAX_LEN = 512
D_MODEL = 64
BATCH = 16384

_NUM_CORES = 1
_NUM_SUBCORES = 16
_NUM_WORKERS = _NUM_CORES * _NUM_SUBCORES
_B_PER_W = BATCH // _NUM_WORKERS

_CHUNKS = 2
_C = _B_PER_W // _CHUNKS

_mesh = plsc.VectorSubcoreMesh(
    core_axis_name="c", subcore_axis_name="s",
    num_cores=_NUM_CORES, num_subcores=_NUM_SUBCORES,
)


@functools.partial(
    pl.kernel,
    out_type=jax.ShapeDtypeStruct((BATCH, 128), jnp.float32),
    mesh=_mesh,
    compiler_params=pltpu.CompilerParams(use_tc_tiling_on_sc=False),
    scratch_types=[
        pltpu.VMEM((_B_PER_W,), jnp.int32),
        pltpu.VMEM((_CHUNKS, _C, D_MODEL), jnp.float32),
        [pltpu.SemaphoreType.DMA] * _CHUNKS,
        [pltpu.SemaphoreType.DMA] * _CHUNKS,
    ],
)
def _sc_gather(table_hbm, idx_hbm, out_hbm, idx_v, rows_v, gsems, wsems):
    wid = lax.axis_index("s") * _NUM_CORES + lax.axis_index("c")
    base = wid * _B_PER_W
    pltpu.sync_copy(idx_hbm.at[pl.ds(base, _B_PER_W)], idx_v)
    gathers = [
        pltpu.async_copy(
            table_hbm.at[idx_v.at[pl.ds(c * _C, _C)]], rows_v.at[c], gsems[c]
        )
        for c in range(_CHUNKS)
    ]
    writes = []
    for c in range(_CHUNKS):
        gathers[c].wait()
        writes.append(
            pltpu.async_copy(
                rows_v.at[c],
                out_hbm.at[pl.ds(base + c * _C, _C), pl.ds(0, D_MODEL)],
                wsems[c],
            )
        )
    for w in writes:
        w.wait()


_TC_ROWS = 8192


def _xpose_body(in_ref, out_ref):
    out_ref[...] = in_ref[:, :D_MODEL].T


_xpose = pl.pallas_call(
    _xpose_body,
    grid=(BATCH // _TC_ROWS,),
    in_specs=[pl.BlockSpec((_TC_ROWS, 128), lambda i: (i, 0))],
    out_specs=pl.BlockSpec((D_MODEL, _TC_ROWS), lambda i: (0, i)),
    out_shape=jax.ShapeDtypeStruct((D_MODEL, BATCH), jnp.float32),
)


def kernel(positions, table):
    staged = _sc_gather(table, positions.astype(jnp.int32))
    return _xpose(staged).T
